# baseline (device time: 350854 ns/iter reference)
import jax
import jax.numpy as jnp
from jax import lax
from jax.experimental import pallas as pl
from jax.experimental.pallas import tpu as pltpu

M_PER = 4096
N = 4096
K_PER = 2048

RCH = 512
NR = M_PER // RCH
HALF_K = K_PER // 2
BCH = 256
NB = HALF_K // BCH

_MESH = pl.DeviceIdType.MESH


LAG = 5


def _v3_body(a_ref, b_ref, o_ref, a_nbr_hbm, b_nbr, aslots, a3slots, stage,
             ysend, fsend, by_recv, bx_recv, a_recv,
             cp_sems, cp3_sems, rb_sems, ot_sems):
    my_x = lax.axis_index("x")
    my_y = lax.axis_index("y")
    nbr_y = (my_x, 1 - my_y)
    nbr_x = (1 - my_x, my_y)

    def b_rows_mine(c):
        return pl.ds(my_x * HALF_K + c * BCH, BCH)

    def b_rows_other(c):
        return pl.ds((1 - my_x) * HALF_K + c * BCH, BCH)

    def b_y_rdma(c):
        return pltpu.make_async_remote_copy(
            src_ref=b_ref.at[b_rows_mine(c)],
            dst_ref=b_nbr.at[b_rows_mine(c)],
            send_sem=ysend.at[c],
            recv_sem=by_recv.at[c],
            device_id=nbr_y,
            device_id_type=_MESH,
        )

    def a_y_rdma(c):
        rows = pl.ds(c * RCH, RCH)
        return pltpu.make_async_remote_copy(
            src_ref=a_ref.at[rows],
            dst_ref=a_nbr_hbm.at[rows],
            send_sem=ysend.at[NB + c],
            recv_sem=a_recv.at[c],
            device_id=nbr_y,
            device_id_type=_MESH,
        )

    def b_x_rdma(c):
        return pltpu.make_async_remote_copy(
            src_ref=b_nbr.at[b_rows_mine(c)],
            dst_ref=b_nbr.at[b_rows_mine(c)],
            send_sem=fsend.at[c],
            recv_sem=bx_recv.at[c],
            device_id=nbr_x,
            device_id_type=_MESH,
        )

    def b_x_wait_rdma(c):
        return pltpu.make_async_remote_copy(
            src_ref=b_nbr.at[b_rows_other(c)],
            dst_ref=b_nbr.at[b_rows_other(c)],
            send_sem=fsend.at[c],
            recv_sem=bx_recv.at[c],
            device_id=nbr_x,
            device_id_type=_MESH,
        )

    def _send_b(c, x):
        b_y_rdma(c).start()
        return x

    lax.fori_loop(0, NB, _send_b, 0)

    def _send_a(c, x):
        a_y_rdma(c).start()
        return x

    lax.fori_loop(0, NR, _send_a, 0)

    def _p3(c):
        s3 = c % 2
        rows = pl.ds(c * RCH, RCH)
        a_y_rdma(c).wait_recv()
        ca = pltpu.make_async_copy(
            a_nbr_hbm.at[rows], a3slots.at[s3], cp3_sems.at[s3])
        ca.start()
        rb = pltpu.make_async_copy(
            o_ref.at[rows], stage.at[s3], rb_sems.at[s3])
        rb.start()
        ca.wait()
        p2 = jnp.dot(
            a3slots[s3], b_nbr[...], preferred_element_type=jnp.float32
        )
        rb.wait()
        stage[s3] = stage[s3] + p2.astype(jnp.bfloat16)
        od = pltpu.make_async_copy(
            stage.at[s3], o_ref.at[rows], ot_sems.at[s3])
        od.start()
        od.wait()

    def _main(r, x):
        s = r % 2
        rows = pl.ds(r * RCH, RCH)
        cp = pltpu.make_async_copy(a_ref.at[rows], aslots.at[s], cp_sems.at[s])
        cp.start()

        @pl.when(r < NB)
        def _():
            b_y_rdma(r).wait_recv()
            b_x_rdma(r).start()

        @pl.when(jnp.logical_and(r >= 1, r <= NB))
        def _():
            b_x_wait_rdma(r - 1).wait_recv()

        cp.wait()
        stage[s] = jnp.dot(
            aslots[s], b_ref[...], preferred_element_type=jnp.float32
        ).astype(jnp.bfloat16)
        od = pltpu.make_async_copy(stage.at[s], o_ref.at[rows], ot_sems.at[s])
        od.start()
        od.wait()

        @pl.when(r >= LAG)
        def _():
            _p3(r - LAG)

        return x

    lax.fori_loop(0, NR, _main, 0)

    def _tail(c, x):
        _p3(c)
        return x

    lax.fori_loop(NR - LAG, NR, _tail, 0)

    def _drain_b(c, x):
        b_y_rdma(c).wait_send()
        return x

    lax.fori_loop(0, NB, _drain_b, 0)

    def _drain_a(c, x):
        a_y_rdma(c).wait_send()
        return x

    lax.fori_loop(0, NR, _drain_a, 0)

    def _drain_f(c, x):
        b_x_rdma(c).wait_send()
        return x

    lax.fori_loop(0, NB, _drain_f, 0)


def kernel(A, B):
    a16 = A.astype(jnp.bfloat16)
    b16 = B.astype(jnp.bfloat16)

    out, _ = pl.pallas_call(
        _v3_body,
        out_shape=[
            jax.ShapeDtypeStruct((M_PER, N), jnp.bfloat16),
            jax.ShapeDtypeStruct((M_PER, K_PER), jnp.bfloat16),
        ],
        in_specs=[
            pl.BlockSpec(memory_space=pl.ANY),
            pl.BlockSpec(memory_space=pltpu.MemorySpace.VMEM),
        ],
        out_specs=[
            pl.BlockSpec(memory_space=pl.ANY),
            pl.BlockSpec(memory_space=pl.ANY),
        ],
        scratch_shapes=[
            pltpu.VMEM((K_PER, N), jnp.bfloat16),
            pltpu.VMEM((2, RCH, K_PER), jnp.bfloat16),
            pltpu.VMEM((2, RCH, K_PER), jnp.bfloat16),
            pltpu.VMEM((2, RCH, N), jnp.bfloat16),
            pltpu.SemaphoreType.DMA((NB + NR,)),
            pltpu.SemaphoreType.DMA((NB,)),
            pltpu.SemaphoreType.DMA((NB,)),
            pltpu.SemaphoreType.DMA((NB,)),
            pltpu.SemaphoreType.DMA((NR,)),
            pltpu.SemaphoreType.DMA((2,)),
            pltpu.SemaphoreType.DMA((2,)),
            pltpu.SemaphoreType.DMA((2,)),
            pltpu.SemaphoreType.DMA((2,)),
        ],
        compiler_params=pltpu.CompilerParams(
            vmem_limit_bytes=60 * 1024 * 1024,
        ),
    )(a16, b16)
    return out


# device time: 312007 ns/iter; 1.1245x vs baseline; 1.1245x over previous
import jax
import jax.numpy as jnp
from jax import lax
from jax.experimental import pallas as pl
from jax.experimental.pallas import tpu as pltpu

M_PER = 4096
N = 4096
K_PER = 2048

RCH = 512
NR = M_PER // RCH
HALF_K = K_PER // 2
BCH = 256
NB = HALF_K // BCH

_MESH = pl.DeviceIdType.MESH


def _v5_body(a_f32, b_f32, o_ref, a_nbr_hbm, a16_hbm, b_nbr, b16,
             aslots, stage, af32s, bf32s,
             ysend, fsend, by_recv, bx_recv, a_recv,
             cp_sems, rb_sems, ot_sems, st_sems):
    my_x = lax.axis_index("x")
    my_y = lax.axis_index("y")
    nbr_y = (my_x, 1 - my_y)
    nbr_x = (1 - my_x, my_y)

    barrier_sem = pltpu.get_barrier_semaphore()
    pl.semaphore_signal(
        barrier_sem, inc=1, device_id=nbr_y, device_id_type=_MESH)
    pl.semaphore_signal(
        barrier_sem, inc=1, device_id=nbr_x, device_id_type=_MESH)
    pl.semaphore_wait(barrier_sem, 2)

    def b_rows_mine(c):
        return pl.ds(my_x * HALF_K + c * BCH, BCH)

    def b_rows_other(c):
        return pl.ds((1 - my_x) * HALF_K + c * BCH, BCH)

    def b_y_rdma(c):
        return pltpu.make_async_remote_copy(
            src_ref=b16.at[b_rows_mine(c)],
            dst_ref=b_nbr.at[b_rows_mine(c)],
            send_sem=ysend.at[c],
            recv_sem=by_recv.at[c],
            device_id=nbr_y,
            device_id_type=_MESH,
        )

    def a_y_rdma(c):
        rows = pl.ds(c * RCH, RCH)
        return pltpu.make_async_remote_copy(
            src_ref=a16_hbm.at[rows],
            dst_ref=a_nbr_hbm.at[rows],
            send_sem=ysend.at[NB + c],
            recv_sem=a_recv.at[c],
            device_id=nbr_y,
            device_id_type=_MESH,
        )

    def b_x_rdma(c):
        return pltpu.make_async_remote_copy(
            src_ref=b_nbr.at[b_rows_mine(c)],
            dst_ref=b_nbr.at[b_rows_mine(c)],
            send_sem=fsend.at[c],
            recv_sem=bx_recv.at[c],
            device_id=nbr_x,
            device_id_type=_MESH,
        )

    def b_x_wait_rdma(c):
        return pltpu.make_async_remote_copy(
            src_ref=b_nbr.at[b_rows_other(c)],
            dst_ref=b_nbr.at[b_rows_other(c)],
            send_sem=fsend.at[c],
            recv_sem=bx_recv.at[c],
            device_id=nbr_x,
            device_id_type=_MESH,
        )

    def _cast_send_b(c, x):
        cb = pltpu.make_async_copy(
            b_f32.at[b_rows_mine(c)], bf32s.at[0], st_sems.at[0])
        cb.start()
        cb.wait()
        b16[b_rows_mine(c)] = bf32s[0].astype(jnp.bfloat16)
        b_y_rdma(c).start()
        return x

    lax.fori_loop(0, NB, _cast_send_b, 0)

    def _cast_send_a(c, x):
        rows = pl.ds(c * RCH, RCH)
        ca = pltpu.make_async_copy(a_f32.at[rows], af32s.at[0], st_sems.at[0])
        ca.start()
        ca.wait()
        s = c % 2
        aslots[s] = af32s[0].astype(jnp.bfloat16)
        ch = pltpu.make_async_copy(aslots.at[s], a16_hbm.at[rows],
                                   cp_sems.at[s])
        ch.start()
        ch.wait()
        a_y_rdma(c).start()
        return x

    lax.fori_loop(0, NR, _cast_send_a, 0)

    def _cast_b2(c, x):
        rows = pl.ds((1 - my_x) * HALF_K + c * BCH, BCH)
        cb = pltpu.make_async_copy(b_f32.at[rows], bf32s.at[0], st_sems.at[0])
        cb.start()
        cb.wait()
        b16[rows] = bf32s[0].astype(jnp.bfloat16)
        return x

    lax.fori_loop(0, NB, _cast_b2, 0)

    def _phase1(r, x):
        s = r % 2
        rows = pl.ds(r * RCH, RCH)
        cp = pltpu.make_async_copy(
            a16_hbm.at[rows], aslots.at[s], cp_sems.at[s])
        cp.start()

        @pl.when(r < NB)
        def _():
            b_y_rdma(r).wait_recv()
            b_x_rdma(r).start()

        cp.wait()
        stage[s] = jnp.dot(
            aslots[s], b16[...], preferred_element_type=jnp.float32
        ).astype(jnp.bfloat16)
        od = pltpu.make_async_copy(stage.at[s], o_ref.at[rows], ot_sems.at[s])
        od.start()
        od.wait()
        return x

    lax.fori_loop(0, NR, _phase1, 0)

    def _phase2(c, x):
        b_x_wait_rdma(c).wait_recv()
        return x

    lax.fori_loop(0, NB, _phase2, 0)

    def _phase3(r, x):
        s = r % 2
        rows = pl.ds(r * RCH, RCH)
        a_y_rdma(r).wait_recv()
        ca = pltpu.make_async_copy(
            a_nbr_hbm.at[rows], aslots.at[s], cp_sems.at[s])
        ca.start()
        rb = pltpu.make_async_copy(o_ref.at[rows], stage.at[s], rb_sems.at[s])
        rb.start()
        ca.wait()
        p2 = jnp.dot(
            aslots[s], b_nbr[...], preferred_element_type=jnp.float32
        )
        rb.wait()
        stage[s] = stage[s] + p2.astype(jnp.bfloat16)
        od = pltpu.make_async_copy(stage.at[s], o_ref.at[rows], ot_sems.at[s])
        od.start()
        od.wait()
        return x

    lax.fori_loop(0, NR, _phase3, 0)

    def _drain_b(c, x):
        b_y_rdma(c).wait_send()
        return x

    lax.fori_loop(0, NB, _drain_b, 0)

    def _drain_a(c, x):
        a_y_rdma(c).wait_send()
        return x

    lax.fori_loop(0, NR, _drain_a, 0)

    def _drain_f(c, x):
        b_x_rdma(c).wait_send()
        return x

    lax.fori_loop(0, NB, _drain_f, 0)


def kernel(A, B):
    out, _, _ = pl.pallas_call(
        _v5_body,
        out_shape=[
            jax.ShapeDtypeStruct((M_PER, N), jnp.bfloat16),
            jax.ShapeDtypeStruct((M_PER, K_PER), jnp.bfloat16),
            jax.ShapeDtypeStruct((M_PER, K_PER), jnp.bfloat16),
        ],
        in_specs=[
            pl.BlockSpec(memory_space=pl.ANY),
            pl.BlockSpec(memory_space=pl.ANY),
        ],
        out_specs=[
            pl.BlockSpec(memory_space=pl.ANY),
            pl.BlockSpec(memory_space=pl.ANY),
            pl.BlockSpec(memory_space=pl.ANY),
        ],
        scratch_shapes=[
            pltpu.VMEM((K_PER, N), jnp.bfloat16),
            pltpu.VMEM((K_PER, N), jnp.bfloat16),
            pltpu.VMEM((2, RCH, K_PER), jnp.bfloat16),
            pltpu.VMEM((2, RCH, N), jnp.bfloat16),
            pltpu.VMEM((1, RCH, K_PER), jnp.float32),
            pltpu.VMEM((1, BCH, N), jnp.float32),
            pltpu.SemaphoreType.DMA((NB + NR,)),
            pltpu.SemaphoreType.DMA((NB,)),
            pltpu.SemaphoreType.DMA((NB,)),
            pltpu.SemaphoreType.DMA((NB,)),
            pltpu.SemaphoreType.DMA((NR,)),
            pltpu.SemaphoreType.DMA((2,)),
            pltpu.SemaphoreType.DMA((2,)),
            pltpu.SemaphoreType.DMA((2,)),
            pltpu.SemaphoreType.DMA((1,)),
        ],
        compiler_params=pltpu.CompilerParams(
            vmem_limit_bytes=62 * 1024 * 1024,
            collective_id=0,
        ),
    )(A, B)
    return out
